# baseline (speedup 1.0000x reference)
import functools

import jax
import jax.numpy as jnp
from jax import lax
from jax.experimental import pallas as pl
from jax.experimental.pallas import tpu as pltpu
from jax.experimental.pallas import tpu_sc as plsc

_N_IMAGES = 150000
_BATCH = 16384
_ROT_D = 36
_TRA_D = 18
_L = 16

_NW = 32
_B_PER_W = _BATCH // _NW
_ROT_WIN = 3
_TRA_WIN = 2
_ROT_GI = _B_PER_W * _ROT_WIN
_TRA_GI = _B_PER_W * _TRA_WIN
_CHUNK = 128
_ROT_OUT_W = _B_PER_W * _ROT_D
_TRA_OUT_W = _B_PER_W * _TRA_D


def _make_gather():
    mesh = plsc.VectorSubcoreMesh(core_axis_name="c", subcore_axis_name="s")

    @functools.partial(
        pl.kernel,
        mesh=mesh,
        compiler_params=pltpu.CompilerParams(
            use_tc_tiling_on_sc=False, needs_layout_passes=False),
        out_type=[
            jax.ShapeDtypeStruct((_BATCH * _ROT_D,), jnp.float32),
            jax.ShapeDtypeStruct((_BATCH * _TRA_D,), jnp.float32),
        ],
        scratch_types=[
            pltpu.VMEM((_B_PER_W,), jnp.int32),
            pltpu.VMEM((_B_PER_W // _L, _L), jnp.int32),
            pltpu.VMEM((_B_PER_W // _L, _L), jnp.int32),
            pltpu.VMEM((_B_PER_W // _L, _L), jnp.int32),
            pltpu.VMEM((_B_PER_W // _L, _L), jnp.int32),
            pltpu.VMEM((_ROT_GI // _CHUNK, _CHUNK), jnp.int32),
            pltpu.VMEM((_TRA_GI // _CHUNK, _CHUNK), jnp.int32),
            pltpu.VMEM((_ROT_GI, _L), jnp.float32),
            pltpu.VMEM((_TRA_GI, _L), jnp.float32),
            pltpu.VMEM((_ROT_OUT_W,), jnp.float32),
            pltpu.VMEM((_TRA_OUT_W,), jnp.float32),
            pltpu.SemaphoreType.DMA,
        ],
    )
    def gather_kernel(rot_hbm, tra_hbm, idx_hbm, rot_out, tra_out,
                      idx_v, v0r, rr, v0t, rt, rot_gi, tra_gi,
                      rot_stage, tra_stage, rot_pack, tra_pack, sem):
        wid = lax.axis_index("s") * 2 + lax.axis_index("c")
        base = wid * _B_PER_W
        iota = lax.iota(jnp.int32, _L)

        pltpu.sync_copy(idx_hbm.at[pl.ds(base, _B_PER_W)], idx_v)

        for c in range(_B_PER_W // _L):
            iv = idx_v[pl.ds(c * _L, _L)]
            nine = iv * 9
            v0r[c] = nine >> 2
            rr[c] = (iv & 3) << 2
            v0t[c] = nine >> 3
            rt[c] = (iv & 7) << 1

        for c in range(_ROT_GI // _L):
            p = iota + (c * _L)
            i = (p * 10923) >> 15
            k = p - ((i << 1) + i)
            val = plsc.load_gather(v0r, [i >> 4, i & 15]) + k
            rot_gi[c // 8, pl.ds((c % 8) * _L, _L)] = val
        for c in range(_TRA_GI // _L):
            p = iota + (c * _L)
            i = p >> 1
            k = p & 1
            val = plsc.load_gather(v0t, [i >> 4, i & 15]) + k
            tra_gi[c // 8, pl.ds((c % 8) * _L, _L)] = val

        copies = []
        for c in range(_ROT_GI // _CHUNK):
            copies.append(pltpu.async_copy(
                rot_hbm.at[rot_gi.at[c]],
                rot_stage.at[pl.ds(c * _CHUNK, _CHUNK)], sem))
        for c in range(_TRA_GI // _CHUNK):
            copies.append(pltpu.async_copy(
                tra_hbm.at[tra_gi.at[c]],
                tra_stage.at[pl.ds(c * _CHUNK, _CHUNK)], sem))
        for cp in copies:
            cp.wait()

        def rot_body(it, _):
            p = iota + it * _L
            q = p >> 2
            j = (q * 7282) >> 16
            t = p + ((j << 3) + (j << 2))
            src = t + plsc.load_gather(rr, [j >> 4, j & 15])
            vals = plsc.load_gather(rot_stage, [src >> 4, src & 15])
            rot_pack[pl.ds(it * _L, _L)] = vals
            return 0

        lax.fori_loop(0, _ROT_OUT_W // _L, rot_body, 0)

        def tra_body(it, _):
            p = iota + it * _L
            q = p >> 1
            j = (q * 7282) >> 16
            t = p + ((j << 4) - (j << 1))
            src = t + plsc.load_gather(rt, [j >> 4, j & 15])
            vals = plsc.load_gather(tra_stage, [src >> 4, src & 15])
            tra_pack[pl.ds(it * _L, _L)] = vals
            return 0

        lax.fori_loop(0, _TRA_OUT_W // _L, tra_body, 0)

        pltpu.sync_copy(rot_pack, rot_out.at[pl.ds(wid * _ROT_OUT_W, _ROT_OUT_W)])
        pltpu.sync_copy(tra_pack, tra_out.at[pl.ds(wid * _TRA_OUT_W, _TRA_OUT_W)])

    return gather_kernel


_GATHER = _make_gather()


def kernel(rotation_per_domain, translation_per_domain, indexes):
    n, d, _ = rotation_per_domain.shape
    rot_view = rotation_per_domain.reshape(n * _ROT_D // _L, _L)
    tra_view = translation_per_domain.reshape(n * _TRA_D // _L, _L)
    idx = indexes.astype(jnp.int32)
    rot_o, tra_o = _GATHER(rot_view, tra_view, idx)
    return (rot_o.reshape(_BATCH, d, 6), tra_o.reshape(_BATCH, d, 3))


# speedup vs baseline: 10.0957x; 10.0957x over previous
import functools

import jax
import jax.numpy as jnp
from jax import lax
from jax.experimental import pallas as pl
from jax.experimental.pallas import tpu as pltpu
from jax.experimental.pallas import tpu_sc as plsc

_N_IMAGES = 150000
_BATCH = 16384
_ROT_D = 36
_TRA_D = 18
_ROT_S = 64
_TRA_S = 32
_L = 16
_W = 128

_NW = 32
_B_PER_W = _BATCH // _NW
_SUB = 128
_NSUB = _B_PER_W // _SUB


def _make_gather():
    mesh = plsc.VectorSubcoreMesh(core_axis_name="c", subcore_axis_name="s")

    @functools.partial(
        pl.kernel,
        mesh=mesh,
        compiler_params=pltpu.CompilerParams(
            use_tc_tiling_on_sc=False, needs_layout_passes=False),
        out_type=[
            jax.ShapeDtypeStruct((6, 8, _BATCH), jnp.float32),
            jax.ShapeDtypeStruct((6, 4, _BATCH), jnp.float32),
        ],
        scratch_types=[
            pltpu.VMEM((_B_PER_W,), jnp.int32),
            pltpu.VMEM((_NSUB, _SUB), jnp.int32),
            pltpu.VMEM((_NSUB, _SUB), jnp.int32),
            pltpu.VMEM((_B_PER_W // _L, _L), jnp.int32),
            pltpu.VMEM((_B_PER_W // _L, _L), jnp.int32),
            pltpu.VMEM((_SUB, _W), jnp.float32),
            pltpu.VMEM((_SUB, _W), jnp.float32),
            pltpu.VMEM((_ROT_D * _B_PER_W,), jnp.float32),
            pltpu.VMEM((_TRA_D * _B_PER_W,), jnp.float32),
            pltpu.SemaphoreType.DMA,
        ],
    )
    def gather_kernel(rot_hbm, tra_hbm, idx_hbm, rot_out, tra_out,
                      idx_v, rot_gi, tra_gi, rr, rt,
                      rot_stage, tra_stage, rot_pack, tra_pack, sem):
        wid = lax.axis_index("s") * 2 + lax.axis_index("c")
        base = wid * _B_PER_W
        iota = lax.iota(jnp.int32, _L)

        pltpu.sync_copy(idx_hbm.at[pl.ds(base, _B_PER_W)], idx_v)

        for c in range(_B_PER_W // _L):
            iv = idx_v[pl.ds(c * _L, _L)]
            rot_gi[c >> 3, pl.ds((c & 7) * _L, _L)] = iv >> 1
            tra_gi[c >> 3, pl.ds((c & 7) * _L, _L)] = iv >> 2
            rr[c] = (iv & 1) << 6
            rt[c] = (iv & 3) << 5

        for b in range(_NSUB):
            cr = pltpu.async_copy(rot_hbm.at[rot_gi.at[b]], rot_stage, sem)
            ct = pltpu.async_copy(tra_hbm.at[tra_gi.at[b]], tra_stage, sem)
            cr.wait()
            ct.wait()

            def rot_body(it, _):
                g = it & 7
                s = it >> 3
                j = iota + g * _L
                src = (j << 7) + s + plsc.load_gather(
                    rr, [(j >> 4) + 8 * b, j & 15])
                vals = plsc.load_gather(rot_stage, [src >> 7, src & 127])
                rot_pack[pl.ds(s * _B_PER_W + b * _SUB + g * _L, _L)] = vals
                return 0

            lax.fori_loop(0, _ROT_D * (_SUB // _L), rot_body, 0)

            def tra_body(it, _):
                g = it & 7
                s = it >> 3
                j = iota + g * _L
                src = (j << 7) + s + plsc.load_gather(
                    rt, [(j >> 4) + 8 * b, j & 15])
                vals = plsc.load_gather(tra_stage, [src >> 7, src & 127])
                tra_pack[pl.ds(s * _B_PER_W + b * _SUB + g * _L, _L)] = vals
                return 0

            lax.fori_loop(0, _TRA_D * (_SUB // _L), tra_body, 0)

        for s in range(_ROT_D):
            d, c = divmod(s, 6)
            pltpu.sync_copy(rot_pack.at[pl.ds(s * _B_PER_W, _B_PER_W)],
                            rot_out.at[d, c, pl.ds(base, _B_PER_W)])
        for s in range(_TRA_D):
            d, c = divmod(s, 3)
            pltpu.sync_copy(tra_pack.at[pl.ds(s * _B_PER_W, _B_PER_W)],
                            tra_out.at[d, c, pl.ds(base, _B_PER_W)])

    return gather_kernel


_GATHER = _make_gather()


def kernel(rotation_per_domain, translation_per_domain, indexes):
    n, d, _ = rotation_per_domain.shape
    rot_rows = jnp.pad(
        rotation_per_domain.reshape(n, _ROT_D), ((0, 0), (0, _ROT_S - _ROT_D))
    ).reshape(n * _ROT_S // _W, _W)
    tra_rows = jnp.pad(
        translation_per_domain.reshape(n, _TRA_D), ((0, 0), (0, _TRA_S - _TRA_D))
    ).reshape(n * _TRA_S // _W, _W)
    idx = indexes.astype(jnp.int32)
    rot_o, tra_o = _GATHER(rot_rows, tra_rows, idx)
    rot = rot_o[:, :6, :].transpose(2, 0, 1)
    tra = tra_o[:, :3, :].transpose(2, 0, 1)
    return (rot, tra)
